# baseline (speedup 1.0000x reference)
import functools

import jax
import jax.numpy as jnp
from jax import lax
from jax.experimental import pallas as pl
from jax.experimental.pallas import tpu as pltpu
from jax.experimental.pallas import tpu_sc as plsc

_BATCH = 4096
_DIM = 64
_BLK = 256
_NSTEP = _BATCH // _BLK
_EPS = 1e-12
_NUM_PAIRS = _BATCH * (_BATCH - 1) // 2

_NC = 2
_NS = 16
_NW = _NC * _NS
_BPW = _BATCH // _NW


def _gather_body(uid_ref, pid_ref, utab_ref, itab_ref, uout_ref, pout_ref,
                 uidx_v, pidx_v, urows_v, prows_v, usem, psem):
    wid = lax.axis_index("s") * _NC + lax.axis_index("c")
    base = wid * _BPW
    pltpu.sync_copy(uid_ref.at[pl.ds(base, _BPW)], uidx_v)
    pltpu.sync_copy(pid_ref.at[pl.ds(base, _BPW)], pidx_v)
    ucopy = pltpu.async_copy(utab_ref.at[uidx_v], urows_v, usem)
    pcopy = pltpu.async_copy(itab_ref.at[pidx_v], prows_v, psem)
    ucopy.wait()
    pltpu.sync_copy(urows_v, uout_ref.at[pl.ds(base, _BPW)])
    pcopy.wait()
    pltpu.sync_copy(prows_v, pout_ref.at[pl.ds(base, _BPW)])


def _make_gather():
    return pl.kernel(
        _gather_body,
        mesh=plsc.VectorSubcoreMesh(core_axis_name="c", subcore_axis_name="s"),
        compiler_params=pltpu.CompilerParams(use_tc_tiling_on_sc=False),
        out_type=[jax.ShapeDtypeStruct((_BATCH, _DIM), jnp.float32)] * 2,
        scratch_types=[
            pltpu.VMEM((_BPW,), jnp.int32),
            pltpu.VMEM((_BPW,), jnp.int32),
            pltpu.VMEM((_BPW, _DIM), jnp.float32),
            pltpu.VMEM((_BPW, _DIM), jnp.float32),
            pltpu.SemaphoreType.DMA,
            pltpu.SemaphoreType.DMA,
        ],
    )


def _loss_body(u_ref, p_ref, out_ref, un_ref, pn_ref, acc_ref):
    i = pl.program_id(0)

    @pl.when(i == 0)
    def _init():
        u = u_ref[...]
        p = p_ref[...]
        un = u / jnp.maximum(jnp.sqrt(jnp.sum(u * u, axis=1, keepdims=True)), _EPS)
        pn = p / jnp.maximum(jnp.sqrt(jnp.sum(p * p, axis=1, keepdims=True)), _EPS)
        un_ref[...] = un
        pn_ref[...] = pn
        d = un - pn
        acc_ref[0] = jnp.sum(d * d)
        ru = jnp.sum(un * un, axis=1, keepdims=True)
        rp = jnp.sum(pn * pn, axis=1, keepdims=True)
        acc_ref[1] = jnp.sum(jnp.exp(-2.0 * jnp.maximum(2.0 - 2.0 * ru, 0.0)))
        acc_ref[2] = jnp.sum(jnp.exp(-2.0 * jnp.maximum(2.0 - 2.0 * rp, 0.0)))
        acc_ref[3] = 0.0
        acc_ref[4] = 0.0

    a_u = un_ref[pl.ds(i * _BLK, _BLK), :]
    g_u = lax.dot_general(a_u, un_ref[...], (((1,), (1,)), ((), ())),
                          preferred_element_type=jnp.float32)
    acc_ref[3] += jnp.sum(jnp.exp(-2.0 * jnp.maximum(2.0 - 2.0 * g_u, 0.0)))
    a_p = pn_ref[pl.ds(i * _BLK, _BLK), :]
    g_p = lax.dot_general(a_p, pn_ref[...], (((1,), (1,)), ((), ())),
                          preferred_element_type=jnp.float32)
    acc_ref[4] += jnp.sum(jnp.exp(-2.0 * jnp.maximum(2.0 - 2.0 * g_p, 0.0)))

    @pl.when(i == _NSTEP - 1)
    def _fin():
        align = acc_ref[0] / _BATCH
        mean_u = (acc_ref[3] - acc_ref[1]) * (0.5 / _NUM_PAIRS)
        mean_p = (acc_ref[4] - acc_ref[2]) * (0.5 / _NUM_PAIRS)
        lu = jnp.log(jnp.full((1, 128), mean_u, jnp.float32))
        lp = jnp.log(jnp.full((1, 128), mean_p, jnp.float32))
        out_ref[...] = align + 0.5 * (lu + lp)


def _loss(u_emb, p_emb):
    out = pl.pallas_call(
        _loss_body,
        grid=(_NSTEP,),
        in_specs=[pl.BlockSpec((_BATCH, _DIM), lambda i: (0, 0))] * 2,
        out_specs=pl.BlockSpec((1, 128), lambda i: (0, 0)),
        out_shape=jax.ShapeDtypeStruct((1, 128), jnp.float32),
        scratch_shapes=[
            pltpu.VMEM((_BATCH, _DIM), jnp.float32),
            pltpu.VMEM((_BATCH, _DIM), jnp.float32),
            pltpu.SMEM((8,), jnp.float32),
        ],
    )(u_emb, p_emb)
    return out[0, 0]


def kernel(user_id, pos_id, neg_id, user_table, item_table):
    u_emb, p_emb = _make_gather()(user_id.astype(jnp.int32), pos_id.astype(jnp.int32),
                                  user_table, item_table)
    return _loss(u_emb, p_emb)
